# baseline (speedup 1.0000x reference)
import jax
import jax.numpy as jnp
from jax import lax
from jax.experimental import pallas as pl
from jax.experimental.pallas import tpu as pltpu
from jax.experimental.pallas import tpu_sc as plsc

NC = 2
NS = 16
L = 16
NW = NC * NS
CH = 20000


G = 10
CHG = 12800


def _sc_edge_filter(si_hbm, ei_hbm, attr_hbm, x_hbm,
                    partials_hbm, xsi_hbm,
                    si_v, ei_v0, ei_v1, attr_g, rows_v, acc,
                    sem0, sem1, semg):
    E = ei_hbm.shape[1]
    nch = E // CHG
    maxk = (nch + NW - 1) // NW
    ng = CHG // (G * L)

    wid = lax.axis_index("s") * NC + lax.axis_index("c")

    acc[...] = jnp.zeros((L,), jnp.float32)
    pltpu.sync_copy(si_hbm, si_v)
    si_vec = si_v[...]

    bufs = (ei_v0, ei_v1)
    sems = (sem0, sem1)

    def start_fetch(ci, buf, sem):
        pltpu.async_copy(ei_hbm.at[:, pl.ds(ci * CHG, CHG)], buf, sem)

    def process_chunk(ci, buf):
        def group_body(g, _):
            gbase = g * (G * L)
            hits = jnp.zeros((L,), jnp.int32)
            for v in range(G):
                dvec = buf[1, pl.ds(gbase + v * L, L)]
                hits = hits + jnp.where(dvec == si_vec, 1, 0)

            @pl.when(jnp.sum(hits) > 0)
            def _():
                def match_body(v, _):
                    voff = gbase + v * L
                    dvec = buf[1, pl.ds(voff, L)]
                    mask = dvec == si_vec
                    nm = jnp.sum(jnp.where(mask, 1, 0))

                    @pl.when(nm > 0)
                    def _():
                        eoff = ci * CHG + voff
                        pltpu.async_copy(
                            attr_hbm.at[pl.ds(eoff, L)], attr_g, semg).wait()
                        avec = plsc.load_gather(
                            attr_g, [lax.iota(jnp.int32, L),
                                     jnp.zeros((L,), jnp.int32)])
                        wv = jnp.where(mask, avec, 0.0)
                        svec = buf[0, pl.ds(voff, L)]
                        for lane in range(L):
                            pltpu.async_copy(
                                x_hbm.at[pl.ds(svec[lane] * L, L)],
                                rows_v.at[lane], semg)
                        for lane in range(L):
                            pltpu.make_async_copy(
                                x_hbm.at[pl.ds(0, L)],
                                rows_v.at[lane], semg).wait()
                        for lane in range(L):
                            acc[...] = acc[...] + wv[lane] * rows_v[lane, :]

                    return 0

                lax.fori_loop(0, G, match_body, 0)

            return 0

        lax.fori_loop(0, ng, group_body, 0)

    c0 = wid
    c1 = wid + NW

    @pl.when(c0 < nch)
    def _():
        start_fetch(c0, bufs[0], sems[0])

    @pl.when(c1 < nch)
    def _():
        start_fetch(c1, bufs[1], sems[1])

    def ring_body(kk, _):
        for ph in range(2):
            c = wid + (2 * kk + ph) * NW

            @pl.when(c < nch)
            def _(c=c, ph=ph):
                pltpu.make_async_copy(
                    ei_hbm.at[:, pl.ds(c * CHG, CHG)],
                    bufs[ph], sems[ph]).wait()
                process_chunk(c, bufs[ph])
                cn = c + 2 * NW

                @pl.when(cn < nch)
                def _():
                    start_fetch(cn, bufs[ph], sems[ph])

        return 0

    lax.fori_loop(0, (maxk + 1) // 2, ring_body, 0)

    pltpu.sync_copy(acc, partials_hbm.at[pl.ds(wid * L, L)])

    @pl.when(wid == 0)
    def _():
        si0 = si_vec[0]
        pltpu.sync_copy(x_hbm.at[pl.ds(si0 * L, L)], rows_v.at[0])
        pltpu.sync_copy(rows_v.at[0], xsi_hbm)


def _sc_call(si_vec, ei, attr, x):
    mesh = plsc.VectorSubcoreMesh(
        core_axis_name="c", subcore_axis_name="s", num_cores=NC, num_subcores=NS)
    return pl.kernel(
        _sc_edge_filter,
        out_type=(
            jax.ShapeDtypeStruct((NW * L,), jnp.float32),
            jax.ShapeDtypeStruct((L,), jnp.float32),
        ),
        mesh=mesh,
        compiler_params=pltpu.CompilerParams(needs_layout_passes=False),
        scratch_types=(
            pltpu.VMEM((L,), jnp.int32),
            pltpu.VMEM((2, CHG), jnp.int32),
            pltpu.VMEM((2, CHG), jnp.int32),
            pltpu.VMEM((L, 1), jnp.float32),
            pltpu.VMEM((L, L), jnp.float32),
            pltpu.VMEM((L,), jnp.float32),
            pltpu.SemaphoreType.DMA,
            pltpu.SemaphoreType.DMA,
            pltpu.SemaphoreType.DMA,
        ),
    )(si_vec, ei, attr, x)


def _tc_dense(partials, xsi, h0, c0, Wn, Ws, bg, WihT, WhhT, bsum,
              W1, b1, W2, b2, xo_out, h_out, c_out):
    dot = lambda a, b: jax.lax.dot(a, b, precision=jax.lax.Precision.DEFAULT,
                                   preferred_element_type=jnp.float32)
    agg = jnp.sum(partials[...], axis=0, keepdims=True)
    xr = xsi[...]
    xg = dot(agg, Wn[...]) + dot(xr, Ws[...]) + bg[...]
    xg = jnp.maximum(xg, 0.0)
    gates = dot(xg, WihT[...]) + dot(h0[...], WhhT[...]) + bsum[...]
    i = jax.nn.sigmoid(gates[:, 0:64])
    f = jax.nn.sigmoid(gates[:, 64:128])
    g = jnp.tanh(gates[:, 128:192])
    o = jax.nn.sigmoid(gates[:, 192:256])
    c1 = f * c0[...] + i * g
    h1 = o * jnp.tanh(c1)
    xcat = jnp.concatenate([xg, h1], axis=1)
    xo = dot(xcat, W1[...]) + b1[...]
    xo_out[...] = dot(xo, W2[...]) + b2[...]
    h_out[...] = h1
    c_out[...] = c1


def kernel(x, edge_index, edge_attr, h, c, state_index,
           W_neigh, W_self, b_gnn, W_ih, W_hh, b_ih, b_hh, W1, b1, W2, b2):
    E = edge_index.shape[1]
    H = W_neigh.shape[1]

    si_vec = jnp.full((L,), jnp.asarray(state_index, jnp.int32), jnp.int32)
    x1 = x.reshape(x.shape[0] * x.shape[1])

    partials, xsi = _sc_call(si_vec, edge_index, edge_attr, x1)
    partials = partials.reshape(NW, L)
    xsi = xsi.reshape(1, L)

    h0 = h.reshape(1, H)
    c0 = c.reshape(1, H)
    bsum = (b_ih + b_hh).reshape(1, 4 * H)

    xo, h1, c1 = pl.pallas_call(
        _tc_dense,
        out_shape=(
            jax.ShapeDtypeStruct((1, 4), jnp.float32),
            jax.ShapeDtypeStruct((1, H), jnp.float32),
            jax.ShapeDtypeStruct((1, H), jnp.float32),
        ),
    )(partials, xsi, h0, c0, W_neigh, W_self, b_gnn.reshape(1, H),
      W_ih.T, W_hh.T, bsum, W1, b1.reshape(1, -1), W2, b2.reshape(1, -1))

    return (xo, h1.reshape(1, 1, H), c1.reshape(1, 1, H))


# speedup vs baseline: 1.0064x; 1.0064x over previous
import jax
import jax.numpy as jnp
from jax import lax
from jax.experimental import pallas as pl
from jax.experimental.pallas import tpu as pltpu
from jax.experimental.pallas import tpu_sc as plsc

NC = 2
NS = 16
L = 16
NW = NC * NS
CH = 20000


G = 10
CHG = 12800


def _sc_edge_filter(si_hbm, ei_hbm, attr_hbm, x_hbm,
                    partials_hbm, xsi_hbm,
                    si_v, ei_v0, ei_v1, attr_g, rows_blk, acc,
                    sem0, sem1, semg):
    E = ei_hbm.shape[1]
    nch = E // CHG
    maxk = (nch + NW - 1) // NW
    ng = CHG // (G * L)

    wid = lax.axis_index("s") * NC + lax.axis_index("c")

    acc[...] = jnp.zeros((L,), jnp.float32)
    pltpu.sync_copy(si_hbm, si_v)
    si_vec = si_v[...]

    bufs = (ei_v0, ei_v1)
    sems = (sem0, sem1)

    def start_fetch(ci, buf, sem):
        pltpu.async_copy(ei_hbm.at[:, pl.ds(ci * CHG, CHG)], buf, sem)

    def process_chunk(ci, buf):
        def group_body(g, _):
            gbase = g * (G * L)
            hits = jnp.zeros((L,), jnp.int32)
            for v in range(G):
                dvec = buf[1, pl.ds(gbase + v * L, L)]
                hits = hits + jnp.where(dvec == si_vec, 1, 0)

            @pl.when(jnp.sum(hits) > 0)
            def _():
                def match_body(v, _):
                    voff = gbase + v * L
                    dvec = buf[1, pl.ds(voff, L)]
                    mask = dvec == si_vec
                    nm = jnp.sum(jnp.where(mask, 1, 0))

                    @pl.when(nm > 0)
                    def _():
                        eoff = ci * CHG + voff
                        pltpu.async_copy(
                            attr_hbm.at[pl.ds(eoff, L)], attr_g, semg).wait()
                        avec = plsc.load_gather(
                            attr_g, [lax.iota(jnp.int32, L),
                                     jnp.zeros((L,), jnp.int32)])
                        wv = jnp.where(mask, avec, 0.0)
                        svec = buf[0, pl.ds(voff, L)]
                        for lane in range(L):
                            rb = (svec[lane] // 8) * 8
                            pltpu.async_copy(
                                x_hbm.at[pl.ds(rb, 8)],
                                rows_blk.at[lane], semg)
                        for lane in range(L):
                            pltpu.make_async_copy(
                                x_hbm.at[pl.ds(0, 8)],
                                rows_blk.at[lane], semg).wait()
                        iot = lax.iota(jnp.int32, L)
                        for lane in range(L):
                            r8 = lax.rem(svec[lane], 8)
                            row = plsc.load_gather(
                                rows_blk,
                                [jnp.full((L,), lane, jnp.int32),
                                 jnp.full((L,), r8, jnp.int32), iot])
                            acc[...] = acc[...] + wv[lane] * row

                    return 0

                lax.fori_loop(0, G, match_body, 0)

            return 0

        lax.fori_loop(0, ng, group_body, 0)

    c0 = wid
    c1 = wid + NW

    @pl.when(c0 < nch)
    def _():
        start_fetch(c0, bufs[0], sems[0])

    @pl.when(c1 < nch)
    def _():
        start_fetch(c1, bufs[1], sems[1])

    def ring_body(kk, _):
        for ph in range(2):
            c = wid + (2 * kk + ph) * NW

            @pl.when(c < nch)
            def _(c=c, ph=ph):
                pltpu.make_async_copy(
                    ei_hbm.at[:, pl.ds(c * CHG, CHG)],
                    bufs[ph], sems[ph]).wait()
                process_chunk(c, bufs[ph])
                cn = c + 2 * NW

                @pl.when(cn < nch)
                def _():
                    start_fetch(cn, bufs[ph], sems[ph])

        return 0

    lax.fori_loop(0, (maxk + 1) // 2, ring_body, 0)

    pltpu.sync_copy(acc, partials_hbm.at[pl.ds(wid * L, L)])

    @pl.when(wid == 0)
    def _():
        si0 = si_vec[0]
        sb = (si0 // 8) * 8
        pltpu.sync_copy(x_hbm.at[pl.ds(sb, 8)], rows_blk.at[0])
        row = plsc.load_gather(
            rows_blk,
            [jnp.zeros((L,), jnp.int32),
             jnp.full((L,), lax.rem(si0, 8), jnp.int32),
             lax.iota(jnp.int32, L)])
        acc[...] = row
        pltpu.sync_copy(acc, xsi_hbm)


def _sc_call(si_vec, ei, attr, x):
    mesh = plsc.VectorSubcoreMesh(
        core_axis_name="c", subcore_axis_name="s", num_cores=NC, num_subcores=NS)
    return pl.kernel(
        _sc_edge_filter,
        out_type=(
            jax.ShapeDtypeStruct((NW * L,), jnp.float32),
            jax.ShapeDtypeStruct((L,), jnp.float32),
        ),
        mesh=mesh,
        compiler_params=pltpu.CompilerParams(needs_layout_passes=False),
        scratch_types=(
            pltpu.VMEM((L,), jnp.int32),
            pltpu.VMEM((2, CHG), jnp.int32),
            pltpu.VMEM((2, CHG), jnp.int32),
            pltpu.VMEM((L, 1), jnp.float32),
            pltpu.VMEM((L, 8, L), jnp.float32),
            pltpu.VMEM((L,), jnp.float32),
            pltpu.SemaphoreType.DMA,
            pltpu.SemaphoreType.DMA,
            pltpu.SemaphoreType.DMA,
        ),
    )(si_vec, ei, attr, x)


def _tc_dense(partials, xsi, h0, c0, Wn, Ws, bg, WihT, WhhT, bsum,
              W1, b1, W2, b2, xo_out, h_out, c_out):
    dot = lambda a, b: jax.lax.dot(a, b, precision=jax.lax.Precision.DEFAULT,
                                   preferred_element_type=jnp.float32)
    agg = jnp.sum(partials[...], axis=0, keepdims=True)
    xr = xsi[...]
    xg = dot(agg, Wn[...]) + dot(xr, Ws[...]) + bg[...]
    xg = jnp.maximum(xg, 0.0)
    gates = dot(xg, WihT[...]) + dot(h0[...], WhhT[...]) + bsum[...]
    i = jax.nn.sigmoid(gates[:, 0:64])
    f = jax.nn.sigmoid(gates[:, 64:128])
    g = jnp.tanh(gates[:, 128:192])
    o = jax.nn.sigmoid(gates[:, 192:256])
    c1 = f * c0[...] + i * g
    h1 = o * jnp.tanh(c1)
    xcat = jnp.concatenate([xg, h1], axis=1)
    xo = dot(xcat, W1[...]) + b1[...]
    xo_out[...] = dot(xo, W2[...]) + b2[...]
    h_out[...] = h1
    c_out[...] = c1


def kernel(x, edge_index, edge_attr, h, c, state_index,
           W_neigh, W_self, b_gnn, W_ih, W_hh, b_ih, b_hh, W1, b1, W2, b2):
    E = edge_index.shape[1]
    H = W_neigh.shape[1]

    si_vec = jnp.full((L,), jnp.asarray(state_index, jnp.int32), jnp.int32)

    partials, xsi = _sc_call(si_vec, edge_index, edge_attr, x)
    partials = partials.reshape(NW, L)
    xsi = xsi.reshape(1, L)

    h0 = h.reshape(1, H)
    c0 = c.reshape(1, H)
    bsum = (b_ih + b_hh).reshape(1, 4 * H)

    xo, h1, c1 = pl.pallas_call(
        _tc_dense,
        out_shape=(
            jax.ShapeDtypeStruct((1, 4), jnp.float32),
            jax.ShapeDtypeStruct((1, H), jnp.float32),
            jax.ShapeDtypeStruct((1, H), jnp.float32),
        ),
    )(partials, xsi, h0, c0, W_neigh, W_self, b_gnn.reshape(1, H),
      W_ih.T, W_hh.T, bsum, W1, b1.reshape(1, -1), W2, b2.reshape(1, -1))

    return (xo, h1.reshape(1, 1, H), c1.reshape(1, 1, H))


# speedup vs baseline: 7.0735x; 7.0286x over previous
import jax
import jax.numpy as jnp
from jax import lax
from jax.experimental import pallas as pl
from jax.experimental.pallas import tpu as pltpu
from jax.experimental.pallas import tpu_sc as plsc

NC = 2
NS = 16
L = 16
NW = NC * NS
CHG = 12800
G = 10
WSL = 6256


def _sc_edge_filter(si_hbm, ei_hbm, attr_hbm, w_hbm,
                    si_v, ei_v0, ei_v1, attr_blk, zeros_v, idx_v, wv_v, w_sh,
                    sem0, sem1, semg):
    E = ei_hbm.shape[1]
    n_nodes = w_hbm.shape[0] // NC
    nch = E // CHG
    maxk = (nch + NW - 1) // NW
    ng = CHG // (G * L)

    cid = lax.axis_index("c")
    sid = lax.axis_index("s")
    wid = sid * NC + cid

    pltpu.sync_copy(si_hbm, si_v)
    si_vec = si_v[...]

    bufs = (ei_v0, ei_v1)
    sems = (sem0, sem1)

    c0 = wid
    c1 = wid + NW

    @pl.when(c0 < nch)
    def _():
        pltpu.async_copy(ei_hbm.at[:, pl.ds(c0 * CHG, CHG)], bufs[0], sems[0])

    @pl.when(c1 < nch)
    def _():
        pltpu.async_copy(ei_hbm.at[:, pl.ds(c1 * CHG, CHG)], bufs[1], sems[1])

    def zero_body(i, _):
        zeros_v[pl.ds(i * L, L)] = jnp.zeros((L,), jnp.float32)
        return 0
    lax.fori_loop(0, WSL // L, zero_body, 0)
    wbase = sid * WSL
    last = n_nodes - 15 * WSL

    @pl.when(sid < NS - 1)
    def _():
        pltpu.sync_copy(zeros_v, w_sh.at[pl.ds(wbase, WSL)])

    @pl.when(sid == NS - 1)
    def _():
        pltpu.sync_copy(zeros_v.at[pl.ds(0, last)], w_sh.at[pl.ds(wbase, last)])

    plsc.subcore_barrier()

    def process_chunk(ci, buf):
        def group_body(g, _):
            gbase = g * (G * L)
            hits = jnp.zeros((L,), jnp.int32)
            for v in range(G):
                dvec = buf[1, pl.ds(gbase + v * L, L)]
                hits = hits + jnp.where(dvec == si_vec, 1, 0)

            @pl.when(jnp.sum(hits) > 0)
            def _():
                def match_body(v, _):
                    voff = gbase + v * L
                    dvec = buf[1, pl.ds(voff, L)]
                    mask = dvec == si_vec
                    nm = jnp.sum(jnp.where(mask, 1, 0))

                    @pl.when(nm > 0)
                    def _():
                        eoff = ci * CHG + voff
                        arow = eoff // 128
                        acol = lax.rem(eoff, 128)
                        arowa = (arow // 8) * 8
                        pltpu.async_copy(
                            attr_hbm.at[pl.ds(arowa, 8)], attr_blk, semg).wait()
                        avec = plsc.load_gather(
                            attr_blk,
                            [jnp.full((L,), arow - arowa, jnp.int32),
                             acol + lax.iota(jnp.int32, L)])
                        wv = jnp.where(mask, avec, 0.0)
                        idx_v[...] = buf[0, pl.ds(voff, L)]
                        wv_v[...] = wv
                        pltpu.sync_copy(wv_v, w_sh.at[idx_v], add=True)

                    return 0

                lax.fori_loop(0, G, match_body, 0)

            return 0

        lax.fori_loop(0, ng, group_body, 0)

    def ring_body(kk, _):
        for ph in range(2):
            c = wid + (2 * kk + ph) * NW

            @pl.when(c < nch)
            def _(c=c, ph=ph):
                pltpu.make_async_copy(
                    ei_hbm.at[:, pl.ds(c * CHG, CHG)],
                    bufs[ph], sems[ph]).wait()
                process_chunk(c, bufs[ph])
                cn = c + 2 * NW

                @pl.when(cn < nch)
                def _():
                    pltpu.async_copy(ei_hbm.at[:, pl.ds(cn * CHG, CHG)],
                                     bufs[ph], sems[ph])

        return 0

    lax.fori_loop(0, (maxk + 1) // 2, ring_body, 0)

    plsc.subcore_barrier()

    @pl.when(sid < NS - 1)
    def _():
        pltpu.sync_copy(w_sh.at[pl.ds(wbase, WSL)], zeros_v)
        pltpu.sync_copy(zeros_v,
                        w_hbm.at[pl.ds(cid * n_nodes + wbase, WSL)])

    @pl.when(sid == NS - 1)
    def _():
        pltpu.sync_copy(w_sh.at[pl.ds(wbase, last)], zeros_v.at[pl.ds(0, last)])
        pltpu.sync_copy(zeros_v.at[pl.ds(0, last)],
                        w_hbm.at[pl.ds(cid * n_nodes + wbase, last)])


def _sc_call(si_vec, ei, attr2, n_nodes):
    mesh = plsc.VectorSubcoreMesh(
        core_axis_name="c", subcore_axis_name="s", num_cores=NC, num_subcores=NS)
    return pl.kernel(
        _sc_edge_filter,
        out_type=jax.ShapeDtypeStruct((NC * n_nodes,), jnp.float32),
        mesh=mesh,
        compiler_params=pltpu.CompilerParams(needs_layout_passes=False),
        scratch_types=(
            pltpu.VMEM((L,), jnp.int32),
            pltpu.VMEM((2, CHG), jnp.int32),
            pltpu.VMEM((2, CHG), jnp.int32),
            pltpu.VMEM((8, 128), jnp.float32),
            pltpu.VMEM((WSL,), jnp.float32),
            pltpu.VMEM((L,), jnp.int32),
            pltpu.VMEM((L,), jnp.float32),
            pltpu.VMEM_SHARED((100000,), jnp.float32),
            pltpu.SemaphoreType.DMA,
            pltpu.SemaphoreType.DMA,
            pltpu.SemaphoreType.DMA,
        ),
    )(si_vec, ei, attr2)


KB = 1024


def _tc_all(si_ref, w_ref, xT_ref, h0, c0, Wn, Ws, bg, WihT, WhhT, bsum,
            W1T, b1, W2T, b2, xo_out, h_out, c_out, acc):
    k = pl.program_id(0)
    nk = pl.num_programs(0)

    @pl.when(k == 0)
    def _():
        acc[...] = jnp.zeros_like(acc)

    si = si_ref[0, 0]
    n = si_ref[0, 1]
    pos = k * KB + lax.broadcasted_iota(jnp.int32, (1, KB), 1)
    valid = pos < n
    oh = jnp.where(pos == si, 1.0, 0.0)
    m = jnp.concatenate(
        [w_ref[...], oh, jnp.zeros((5, KB), jnp.float32)], axis=0)
    m = jnp.where(valid, m, 0.0)
    xb = jnp.where(valid, xT_ref[...], 0.0)
    acc[...] += lax.dot_general(
        m, xb, (((1,), (1,)), ((), ())),
        precision=jax.lax.Precision.HIGHEST,
        preferred_element_type=jnp.float32)

    @pl.when(k == nk - 1)
    def _():
        dotn = lambda a, b: lax.dot_general(
            a, b, (((1,), (0,)), ((), ())),
            preferred_element_type=jnp.float32)
        dott = lambda a, b: lax.dot_general(
            a, b, (((1,), (1,)), ((), ())),
            preferred_element_type=jnp.float32)
        a = acc[...]
        agg = a[0:1, :] + a[1:2, :]
        xsi = a[2:3, :]
        xg = dotn(agg, Wn[...]) + dotn(xsi, Ws[...]) + bg[...]
        xg = jnp.maximum(xg, 0.0)
        gates = dotn(xg, WihT[...]) + dotn(h0[...], WhhT[...]) + bsum[...]
        i = jax.nn.sigmoid(gates[:, 0:64])
        f = jax.nn.sigmoid(gates[:, 64:128])
        g = jnp.tanh(gates[:, 128:192])
        o = jax.nn.sigmoid(gates[:, 192:256])
        c1 = f * c0[...] + i * g
        h1 = o * jnp.tanh(c1)
        xcat = jnp.concatenate([xg, h1], axis=1)
        xo = dott(xcat, W1T[...]) + b1[...]
        xo_out[...] = dott(xo, W2T[...]) + b2[...]
        h_out[...] = h1
        c_out[...] = c1


def kernel(x, edge_index, edge_attr, h, c, state_index,
           W_neigh, W_self, b_gnn, W_ih, W_hh, b_ih, b_hh, W1, b1, W2, b2):
    n, nf = x.shape
    E = edge_index.shape[1]
    H = W_neigh.shape[1]

    si = jnp.asarray(state_index, jnp.int32)
    si_vec = jnp.full((L,), si, jnp.int32)
    attr2 = edge_attr.reshape(E // 128, 128)

    wparts = _sc_call(si_vec, edge_index, attr2, n).reshape(NC, n)

    nkb = (n + KB - 1) // KB
    full = lambda s: pl.BlockSpec(s, lambda k: (0, 0))
    xo, h1, c1 = pl.pallas_call(
        _tc_all,
        grid=(nkb,),
        in_specs=[
            pl.BlockSpec(memory_space=pltpu.SMEM),
            pl.BlockSpec((NC, KB), lambda k: (0, k)),
            pl.BlockSpec((nf, KB), lambda k: (0, k)),
            full((1, H)), full((1, H)),
            full((nf, H)), full((nf, H)), full((1, H)),
            full((H, 4 * H)), full((H, 4 * H)), full((1, 4 * H)),
            full((W1.shape[1], W1.shape[0])),
            full((1, W1.shape[1])),
            full((W2.shape[1], W2.shape[0])), full((1, W2.shape[1])),
        ],
        out_specs=[full((1, 4)), full((1, H)), full((1, H))],
        out_shape=(
            jax.ShapeDtypeStruct((1, 4), jnp.float32),
            jax.ShapeDtypeStruct((1, H), jnp.float32),
            jax.ShapeDtypeStruct((1, H), jnp.float32),
        ),
        scratch_shapes=[pltpu.VMEM((8, nf), jnp.float32)],
    )(jnp.stack([si, jnp.int32(n)]).reshape(1, 2), wparts, x.T,
      h.reshape(1, H), c.reshape(1, H),
      W_neigh, W_self, b_gnn.reshape(1, H), W_ih.T, W_hh.T,
      (b_ih + b_hh).reshape(1, 4 * H), W1.T, b1.reshape(1, -1),
      W2.T, b2.reshape(1, -1))

    return (xo, h1.reshape(1, 1, H), c1.reshape(1, 1, H))


# speedup vs baseline: 12.9462x; 1.8302x over previous
import jax
import jax.numpy as jnp
from jax import lax
from jax.experimental import pallas as pl
from jax.experimental.pallas import tpu as pltpu
from jax.experimental.pallas import tpu_sc as plsc

NC = 2
NS = 16
L = 16
NW = NC * NS
CHG = 12800
G = 16
WSL = 6256


def _sc_edge_filter(si_hbm, ei_hbm, attr_hbm, w_hbm,
                    si_v, ei_v0, ei_v1, attr_blk, zeros_v, idx_v, wv_v, w_sh,
                    sem0, sem1, semg):
    E = ei_hbm.shape[1]
    n_nodes = w_hbm.shape[0] // NC
    nch = E // CHG
    maxk = (nch + NW - 1) // NW
    ng = CHG // (G * L)

    cid = lax.axis_index("c")
    sid = lax.axis_index("s")
    wid = sid * NC + cid

    pltpu.sync_copy(si_hbm, si_v)
    si_vec = si_v[...]

    bufs = (ei_v0, ei_v1)
    sems = (sem0, sem1)

    c0 = wid
    c1 = wid + NW

    @pl.when(c0 < nch)
    def _():
        pltpu.async_copy(ei_hbm.at[:, pl.ds(c0 * CHG, CHG)], bufs[0], sems[0])

    @pl.when(c1 < nch)
    def _():
        pltpu.async_copy(ei_hbm.at[:, pl.ds(c1 * CHG, CHG)], bufs[1], sems[1])

    def zero_body(i, _):
        zeros_v[pl.ds(i * L, L)] = jnp.zeros((L,), jnp.float32)
        return 0
    lax.fori_loop(0, WSL // L, zero_body, 0)
    wbase = sid * WSL
    last = n_nodes - 15 * WSL

    @pl.when(sid < NS - 1)
    def _():
        pltpu.sync_copy(zeros_v, w_sh.at[pl.ds(wbase, WSL)])

    @pl.when(sid == NS - 1)
    def _():
        pltpu.sync_copy(zeros_v.at[pl.ds(0, last)], w_sh.at[pl.ds(wbase, last)])

    plsc.subcore_barrier()

    def process_chunk(ci, buf):
        def group_body(g, _):
            gbase = g * (G * L)
            hits = jnp.zeros((L,), jnp.int32)
            for v in range(G):
                dvec = buf[1, pl.ds(gbase + v * L, L)]
                hits = hits + jnp.where(dvec == si_vec, 1, 0)

            @pl.when(jnp.sum(hits) > 0)
            def _():
                def match_body(v, _):
                    voff = gbase + v * L
                    dvec = buf[1, pl.ds(voff, L)]
                    mask = dvec == si_vec
                    nm = jnp.sum(jnp.where(mask, 1, 0))

                    @pl.when(nm > 0)
                    def _():
                        eoff = ci * CHG + voff
                        arow = eoff // 128
                        acol = lax.rem(eoff, 128)
                        arowa = (arow // 8) * 8
                        pltpu.async_copy(
                            attr_hbm.at[pl.ds(arowa, 8)], attr_blk, semg).wait()
                        avec = plsc.load_gather(
                            attr_blk,
                            [jnp.full((L,), arow - arowa, jnp.int32),
                             acol + lax.iota(jnp.int32, L)])
                        wv = jnp.where(mask, avec, 0.0)
                        idx_v[...] = buf[0, pl.ds(voff, L)]
                        wv_v[...] = wv
                        pltpu.sync_copy(wv_v, w_sh.at[idx_v], add=True)

                    return 0

                lax.fori_loop(0, G, match_body, 0)

            return 0

        lax.fori_loop(0, ng, group_body, 0)

    def ring_body(kk, _):
        for ph in range(2):
            c = wid + (2 * kk + ph) * NW

            @pl.when(c < nch)
            def _(c=c, ph=ph):
                pltpu.make_async_copy(
                    ei_hbm.at[:, pl.ds(c * CHG, CHG)],
                    bufs[ph], sems[ph]).wait()
                process_chunk(c, bufs[ph])
                cn = c + 2 * NW

                @pl.when(cn < nch)
                def _():
                    pltpu.async_copy(ei_hbm.at[:, pl.ds(cn * CHG, CHG)],
                                     bufs[ph], sems[ph])

        return 0

    lax.fori_loop(0, (maxk + 1) // 2, ring_body, 0)

    plsc.subcore_barrier()

    @pl.when(sid < NS - 1)
    def _():
        pltpu.sync_copy(w_sh.at[pl.ds(wbase, WSL)], zeros_v)
        pltpu.sync_copy(zeros_v,
                        w_hbm.at[pl.ds(cid * n_nodes + wbase, WSL)])

    @pl.when(sid == NS - 1)
    def _():
        pltpu.sync_copy(w_sh.at[pl.ds(wbase, last)], zeros_v.at[pl.ds(0, last)])
        pltpu.sync_copy(zeros_v.at[pl.ds(0, last)],
                        w_hbm.at[pl.ds(cid * n_nodes + wbase, last)])


def _sc_call(si_vec, ei, attr2, n_nodes):
    mesh = plsc.VectorSubcoreMesh(
        core_axis_name="c", subcore_axis_name="s", num_cores=NC, num_subcores=NS)
    return pl.kernel(
        _sc_edge_filter,
        out_type=jax.ShapeDtypeStruct((NC * n_nodes,), jnp.float32),
        mesh=mesh,
        compiler_params=pltpu.CompilerParams(needs_layout_passes=False),
        scratch_types=(
            pltpu.VMEM((L,), jnp.int32),
            pltpu.VMEM((2, CHG), jnp.int32),
            pltpu.VMEM((2, CHG), jnp.int32),
            pltpu.VMEM((8, 128), jnp.float32),
            pltpu.VMEM((WSL,), jnp.float32),
            pltpu.VMEM((L,), jnp.int32),
            pltpu.VMEM((L,), jnp.float32),
            pltpu.VMEM_SHARED((100000,), jnp.float32),
            pltpu.SemaphoreType.DMA,
            pltpu.SemaphoreType.DMA,
            pltpu.SemaphoreType.DMA,
        ),
    )(si_vec, ei, attr2)


KB = 8192


def _tc_all(si_ref, w_ref, xT_ref, h0, c0, Wn, Ws, bg, WihT, WhhT, bsum,
            W1T, b1, W2T, b2, xo_out, h_out, c_out, acc):
    k = pl.program_id(0)
    nk = pl.num_programs(0)

    @pl.when(k == 0)
    def _():
        acc[...] = jnp.zeros_like(acc)

    si = si_ref[0, 0]
    n = si_ref[0, 1]
    pos = k * KB + lax.broadcasted_iota(jnp.int32, (1, KB), 1)
    valid = pos < n
    oh = jnp.where(pos == si, 1.0, 0.0)
    m = jnp.concatenate(
        [w_ref[...], oh, jnp.zeros((5, KB), jnp.float32)], axis=0)
    m = jnp.where(valid, m, 0.0)
    xb = jnp.where(valid, xT_ref[...], 0.0)
    acc[...] += lax.dot_general(
        m, xb, (((1,), (1,)), ((), ())),
        precision=jax.lax.Precision.HIGHEST,
        preferred_element_type=jnp.float32)

    @pl.when(k == nk - 1)
    def _():
        dotn = lambda a, b: lax.dot_general(
            a, b, (((1,), (0,)), ((), ())),
            preferred_element_type=jnp.float32)
        dott = lambda a, b: lax.dot_general(
            a, b, (((1,), (1,)), ((), ())),
            preferred_element_type=jnp.float32)
        a = acc[...]
        agg = a[0:1, :] + a[1:2, :]
        xsi = a[2:3, :]
        xg = dotn(agg, Wn[...]) + dotn(xsi, Ws[...]) + bg[...]
        xg = jnp.maximum(xg, 0.0)
        gates = dotn(xg, WihT[...]) + dotn(h0[...], WhhT[...]) + bsum[...]
        i = jax.nn.sigmoid(gates[:, 0:64])
        f = jax.nn.sigmoid(gates[:, 64:128])
        g = jnp.tanh(gates[:, 128:192])
        o = jax.nn.sigmoid(gates[:, 192:256])
        c1 = f * c0[...] + i * g
        h1 = o * jnp.tanh(c1)
        xcat = jnp.concatenate([xg, h1], axis=1)
        xo = dott(xcat, W1T[...]) + b1[...]
        xo_out[...] = dott(xo, W2T[...]) + b2[...]
        h_out[...] = h1
        c_out[...] = c1


def kernel(x, edge_index, edge_attr, h, c, state_index,
           W_neigh, W_self, b_gnn, W_ih, W_hh, b_ih, b_hh, W1, b1, W2, b2):
    n, nf = x.shape
    E = edge_index.shape[1]
    H = W_neigh.shape[1]

    si = jnp.asarray(state_index, jnp.int32)
    si_vec = jnp.full((L,), si, jnp.int32)
    attr2 = edge_attr.reshape(E // 128, 128)

    wparts = _sc_call(si_vec, edge_index, attr2, n).reshape(NC, n)

    nkb = (n + KB - 1) // KB
    full = lambda s: pl.BlockSpec(s, lambda k: (0, 0))
    xo, h1, c1 = pl.pallas_call(
        _tc_all,
        grid=(nkb,),
        in_specs=[
            pl.BlockSpec(memory_space=pltpu.SMEM),
            pl.BlockSpec((NC, KB), lambda k: (0, k)),
            pl.BlockSpec((nf, KB), lambda k: (0, k)),
            full((1, H)), full((1, H)),
            full((nf, H)), full((nf, H)), full((1, H)),
            full((H, 4 * H)), full((H, 4 * H)), full((1, 4 * H)),
            full((W1.shape[1], W1.shape[0])),
            full((1, W1.shape[1])),
            full((W2.shape[1], W2.shape[0])), full((1, W2.shape[1])),
        ],
        out_specs=[full((1, 4)), full((1, H)), full((1, H))],
        out_shape=(
            jax.ShapeDtypeStruct((1, 4), jnp.float32),
            jax.ShapeDtypeStruct((1, H), jnp.float32),
            jax.ShapeDtypeStruct((1, H), jnp.float32),
        ),
        scratch_shapes=[pltpu.VMEM((8, nf), jnp.float32)],
    )(jnp.stack([si, jnp.int32(n)]).reshape(1, 2), wparts, x.T,
      h.reshape(1, H), c.reshape(1, H),
      W_neigh, W_self, b_gnn.reshape(1, H), W_ih.T, W_hh.T,
      (b_ih + b_hh).reshape(1, 4 * H), W1.T, b1.reshape(1, -1),
      W2.T, b2.reshape(1, -1))

    return (xo, h1.reshape(1, 1, H), c1.reshape(1, 1, H))


# speedup vs baseline: 13.1287x; 1.0141x over previous
import jax
import jax.numpy as jnp
from jax import lax
from jax.experimental import pallas as pl
from jax.experimental.pallas import tpu as pltpu
from jax.experimental.pallas import tpu_sc as plsc

NC = 2
NS = 16
L = 16
NW = NC * NS
CHG = 12800
G = 16
WSL = 6256


def _sc_edge_filter(si_hbm, ei_hbm, attr_hbm, w_hbm,
                    si_v, ei_v0, ei_v1, attr_blk, zeros_v, idx_v, wv_v, w_sh,
                    sem0, sem1, semg):
    E = ei_hbm.shape[1]
    n_nodes = w_hbm.shape[0] // NC
    nch = E // CHG
    maxk = (nch + NW - 1) // NW
    ng = CHG // (G * L)

    cid = lax.axis_index("c")
    sid = lax.axis_index("s")
    wid = sid * NC + cid

    pltpu.sync_copy(si_hbm, si_v)
    si_vec = si_v[...]

    bufs = (ei_v0, ei_v1)
    sems = (sem0, sem1)

    c0 = wid
    c1 = wid + NW

    @pl.when(c0 < nch)
    def _():
        pltpu.async_copy(ei_hbm.at[:, pl.ds(c0 * CHG, CHG)], bufs[0], sems[0])

    @pl.when(c1 < nch)
    def _():
        pltpu.async_copy(ei_hbm.at[:, pl.ds(c1 * CHG, CHG)], bufs[1], sems[1])

    def zero_body(i, _):
        zeros_v[pl.ds(i * L, L)] = jnp.zeros((L,), jnp.float32)
        return 0
    lax.fori_loop(0, WSL // L, zero_body, 0)
    wbase = sid * WSL
    last = n_nodes - 15 * WSL

    @pl.when(sid < NS - 1)
    def _():
        pltpu.sync_copy(zeros_v, w_sh.at[pl.ds(wbase, WSL)])

    @pl.when(sid == NS - 1)
    def _():
        pltpu.sync_copy(zeros_v.at[pl.ds(0, last)], w_sh.at[pl.ds(wbase, last)])

    plsc.subcore_barrier()

    def process_chunk(ci, buf):
        def group_body(g, _):
            gbase = g * (G * L)
            hits = jnp.zeros((L,), jnp.int32)
            for v in range(G):
                dvec = buf[1, pl.ds(gbase + v * L, L)]
                hits = hits + jnp.where(dvec == si_vec, 1, 0)

            @pl.when(jnp.sum(hits) > 0)
            def _():
                def match_body(v, _):
                    voff = gbase + v * L
                    dvec = buf[1, pl.ds(voff, L)]
                    mask = dvec == si_vec
                    nm = jnp.sum(jnp.where(mask, 1, 0))

                    @pl.when(nm > 0)
                    def _():
                        eoff = ci * CHG + voff
                        arow = eoff // 128
                        acol = lax.rem(eoff, 128)
                        arowa = (arow // 8) * 8
                        pltpu.async_copy(
                            attr_hbm.at[pl.ds(arowa, 8)], attr_blk, semg).wait()
                        avec = plsc.load_gather(
                            attr_blk,
                            [jnp.full((L,), arow - arowa, jnp.int32),
                             acol + lax.iota(jnp.int32, L)])
                        wv = jnp.where(mask, avec, 0.0)
                        idx_v[...] = buf[0, pl.ds(voff, L)]
                        wv_v[...] = wv
                        pltpu.sync_copy(wv_v, w_sh.at[idx_v], add=True)

                    return 0

                lax.fori_loop(0, G, match_body, 0)

            return 0

        lax.fori_loop(0, ng, group_body, 0)

    def ring_body(kk, _):
        for ph in range(2):
            c = wid + (2 * kk + ph) * NW

            @pl.when(c < nch)
            def _(c=c, ph=ph):
                pltpu.make_async_copy(
                    ei_hbm.at[:, pl.ds(c * CHG, CHG)],
                    bufs[ph], sems[ph]).wait()
                process_chunk(c, bufs[ph])
                cn = c + 2 * NW

                @pl.when(cn < nch)
                def _():
                    pltpu.async_copy(ei_hbm.at[:, pl.ds(cn * CHG, CHG)],
                                     bufs[ph], sems[ph])

        return 0

    lax.fori_loop(0, (maxk + 1) // 2, ring_body, 0)

    plsc.subcore_barrier()

    @pl.when(sid < NS - 1)
    def _():
        pltpu.sync_copy(w_sh.at[pl.ds(wbase, WSL)], zeros_v)
        pltpu.sync_copy(zeros_v,
                        w_hbm.at[pl.ds(cid * n_nodes + wbase, WSL)])

    @pl.when(sid == NS - 1)
    def _():
        pltpu.sync_copy(w_sh.at[pl.ds(wbase, last)], zeros_v.at[pl.ds(0, last)])
        pltpu.sync_copy(zeros_v.at[pl.ds(0, last)],
                        w_hbm.at[pl.ds(cid * n_nodes + wbase, last)])


def _sc_call(si_vec, ei, attr2, n_nodes):
    mesh = plsc.VectorSubcoreMesh(
        core_axis_name="c", subcore_axis_name="s", num_cores=NC, num_subcores=NS)
    return pl.kernel(
        _sc_edge_filter,
        out_type=jax.ShapeDtypeStruct((NC * n_nodes,), jnp.float32),
        mesh=mesh,
        compiler_params=pltpu.CompilerParams(needs_layout_passes=False),
        scratch_types=(
            pltpu.VMEM((L,), jnp.int32),
            pltpu.VMEM((2, CHG), jnp.int32),
            pltpu.VMEM((2, CHG), jnp.int32),
            pltpu.VMEM((8, 128), jnp.float32),
            pltpu.VMEM((WSL,), jnp.float32),
            pltpu.VMEM((L,), jnp.int32),
            pltpu.VMEM((L,), jnp.float32),
            pltpu.VMEM_SHARED((100000,), jnp.float32),
            pltpu.SemaphoreType.DMA,
            pltpu.SemaphoreType.DMA,
            pltpu.SemaphoreType.DMA,
        ),
    )(si_vec, ei, attr2)


KB = 8192


def _tc_all(si_ref, w_ref, xT_ref, h0, c0, Wn, Ws, bg, WihT, WhhT, bsum,
            W1T, b1, W2T, b2, xo_out, h_out, c_out):
    si = si_ref[0, 0]
    n = xT_ref.shape[1]
    pos = lax.broadcasted_iota(jnp.int32, (1, n), 1)
    oh = jnp.where(pos == si, 1.0, 0.0)
    m = jnp.concatenate([w_ref[...], oh], axis=0)
    a = lax.dot_general(
        m, xT_ref[...], (((1,), (1,)), ((), ())),
        precision=jax.lax.Precision.HIGHEST,
        preferred_element_type=jnp.float32)
    dotn = lambda a, b: lax.dot_general(
        a, b, (((1,), (0,)), ((), ())),
        preferred_element_type=jnp.float32)
    dott = lambda a, b: lax.dot_general(
        a, b, (((1,), (1,)), ((), ())),
        preferred_element_type=jnp.float32)
    agg = a[0:1, :] + a[1:2, :]
    xsi = a[2:3, :]
    xg = dotn(agg, Wn[...]) + dotn(xsi, Ws[...]) + bg[...]
    xg = jnp.maximum(xg, 0.0)
    gates = dotn(xg, WihT[...]) + dotn(h0[...], WhhT[...]) + bsum[...]
    i = jax.nn.sigmoid(gates[:, 0:64])
    f = jax.nn.sigmoid(gates[:, 64:128])
    g = jnp.tanh(gates[:, 128:192])
    o = jax.nn.sigmoid(gates[:, 192:256])
    c1 = f * c0[...] + i * g
    h1 = o * jnp.tanh(c1)
    xcat = jnp.concatenate([xg, h1], axis=1)
    xo = dott(xcat, W1T[...]) + b1[...]
    xo_out[...] = dott(xo, W2T[...]) + b2[...]
    h_out[...] = h1
    c_out[...] = c1


def kernel(x, edge_index, edge_attr, h, c, state_index,
           W_neigh, W_self, b_gnn, W_ih, W_hh, b_ih, b_hh, W1, b1, W2, b2):
    n, nf = x.shape
    E = edge_index.shape[1]
    H = W_neigh.shape[1]

    si = jnp.asarray(state_index, jnp.int32)
    si_vec = jnp.full((L,), si, jnp.int32)
    attr2 = edge_attr.reshape(E // 128, 128)

    wparts = _sc_call(si_vec, edge_index, attr2, n).reshape(NC, n)

    smem_spec = pl.BlockSpec(memory_space=pltpu.SMEM)
    xo, h1, c1 = pl.pallas_call(
        _tc_all,
        in_specs=[smem_spec] + [pl.BlockSpec()] * 14,
        out_shape=(
            jax.ShapeDtypeStruct((1, 4), jnp.float32),
            jax.ShapeDtypeStruct((1, H), jnp.float32),
            jax.ShapeDtypeStruct((1, H), jnp.float32),
        ),
    )(si.reshape(1, 1), wparts, x.T,
      h.reshape(1, H), c.reshape(1, H),
      W_neigh, W_self, b_gnn.reshape(1, H), W_ih.T, W_hh.T,
      (b_ih + b_hh).reshape(1, 4 * H), W1.T, b1.reshape(1, -1),
      W2.T, b2.reshape(1, -1))

    return (xo, h1.reshape(1, 1, H), c1.reshape(1, 1, H))


# speedup vs baseline: 13.6707x; 1.0413x over previous
import jax
import jax.numpy as jnp
from jax import lax
from jax.experimental import pallas as pl
from jax.experimental.pallas import tpu as pltpu
from jax.experimental.pallas import tpu_sc as plsc

NC = 2
NS = 16
L = 16
NW = NC * NS
CHG = 25600
G = 32
WSL = 6256


def _sc_edge_filter(si_hbm, ei_hbm, attr_hbm, w_hbm,
                    si_v, ei_v0, ei_v1, attr_blk, zeros_v, idx_v, wv_v, w_sh,
                    sem0, sem1, semg):
    E = ei_hbm.shape[1]
    n_nodes = w_hbm.shape[0] // NC
    nch = E // CHG
    maxk = (nch + NW - 1) // NW
    ng = CHG // (G * L)

    cid = lax.axis_index("c")
    sid = lax.axis_index("s")
    wid = sid * NC + cid

    pltpu.sync_copy(si_hbm, si_v)
    si_vec = si_v[...]

    bufs = (ei_v0, ei_v1)
    sems = (sem0, sem1)

    c0 = wid
    c1 = wid + NW

    @pl.when(c0 < nch)
    def _():
        pltpu.async_copy(ei_hbm.at[:, pl.ds(c0 * CHG, CHG)], bufs[0], sems[0])

    @pl.when(c1 < nch)
    def _():
        pltpu.async_copy(ei_hbm.at[:, pl.ds(c1 * CHG, CHG)], bufs[1], sems[1])

    def zero_body(i, _):
        zeros_v[pl.ds(i * L, L)] = jnp.zeros((L,), jnp.float32)
        return 0
    lax.fori_loop(0, WSL // L, zero_body, 0)
    wbase = sid * WSL
    last = n_nodes - 15 * WSL

    @pl.when(sid < NS - 1)
    def _():
        pltpu.sync_copy(zeros_v, w_sh.at[pl.ds(wbase, WSL)])

    @pl.when(sid == NS - 1)
    def _():
        pltpu.sync_copy(zeros_v.at[pl.ds(0, last)], w_sh.at[pl.ds(wbase, last)])

    plsc.subcore_barrier()

    def process_chunk(ci, buf):
        def group_body(g, _):
            gbase = g * (G * L)
            hitm = buf[1, pl.ds(gbase, L)] == si_vec
            for v in range(1, G):
                dvec = buf[1, pl.ds(gbase + v * L, L)]
                hitm = hitm | (dvec == si_vec)

            @pl.when(jnp.sum(jnp.where(hitm, 1, 0)) > 0)
            def _():
                def match_body(v, _):
                    voff = gbase + v * L
                    dvec = buf[1, pl.ds(voff, L)]
                    mask = dvec == si_vec
                    nm = jnp.sum(jnp.where(mask, 1, 0))

                    @pl.when(nm > 0)
                    def _():
                        eoff = ci * CHG + voff
                        arow = eoff // 128
                        acol = lax.rem(eoff, 128)
                        arowa = (arow // 8) * 8
                        pltpu.async_copy(
                            attr_hbm.at[pl.ds(arowa, 8)], attr_blk, semg).wait()
                        avec = plsc.load_gather(
                            attr_blk,
                            [jnp.full((L,), arow - arowa, jnp.int32),
                             acol + lax.iota(jnp.int32, L)])
                        wv = jnp.where(mask, avec, 0.0)
                        idx_v[...] = buf[0, pl.ds(voff, L)]
                        wv_v[...] = wv
                        pltpu.sync_copy(wv_v, w_sh.at[idx_v], add=True)

                    return 0

                lax.fori_loop(0, G, match_body, 0)

            return 0

        lax.fori_loop(0, ng, group_body, 0)

    def ring_body(kk, _):
        for ph in range(2):
            c = wid + (2 * kk + ph) * NW

            @pl.when(c < nch)
            def _(c=c, ph=ph):
                pltpu.make_async_copy(
                    ei_hbm.at[:, pl.ds(c * CHG, CHG)],
                    bufs[ph], sems[ph]).wait()
                process_chunk(c, bufs[ph])
                cn = c + 2 * NW

                @pl.when(cn < nch)
                def _():
                    pltpu.async_copy(ei_hbm.at[:, pl.ds(cn * CHG, CHG)],
                                     bufs[ph], sems[ph])

        return 0

    lax.fori_loop(0, (maxk + 1) // 2, ring_body, 0)

    plsc.subcore_barrier()

    @pl.when(sid < NS - 1)
    def _():
        pltpu.sync_copy(w_sh.at[pl.ds(wbase, WSL)], zeros_v)
        pltpu.sync_copy(zeros_v,
                        w_hbm.at[pl.ds(cid * n_nodes + wbase, WSL)])

    @pl.when(sid == NS - 1)
    def _():
        pltpu.sync_copy(w_sh.at[pl.ds(wbase, last)], zeros_v.at[pl.ds(0, last)])
        pltpu.sync_copy(zeros_v.at[pl.ds(0, last)],
                        w_hbm.at[pl.ds(cid * n_nodes + wbase, last)])


def _sc_call(si_vec, ei, attr2, n_nodes):
    mesh = plsc.VectorSubcoreMesh(
        core_axis_name="c", subcore_axis_name="s", num_cores=NC, num_subcores=NS)
    return pl.kernel(
        _sc_edge_filter,
        out_type=jax.ShapeDtypeStruct((NC * n_nodes,), jnp.float32),
        mesh=mesh,
        compiler_params=pltpu.CompilerParams(needs_layout_passes=False),
        scratch_types=(
            pltpu.VMEM((L,), jnp.int32),
            pltpu.VMEM((2, CHG), jnp.int32),
            pltpu.VMEM((2, CHG), jnp.int32),
            pltpu.VMEM((8, 128), jnp.float32),
            pltpu.VMEM((WSL,), jnp.float32),
            pltpu.VMEM((L,), jnp.int32),
            pltpu.VMEM((L,), jnp.float32),
            pltpu.VMEM_SHARED((100000,), jnp.float32),
            pltpu.SemaphoreType.DMA,
            pltpu.SemaphoreType.DMA,
            pltpu.SemaphoreType.DMA,
        ),
    )(si_vec, ei, attr2)


KB = 8192


def _tc_all(si_ref, w_ref, xT_ref, h0, c0, Wn, Ws, bg, WihT, WhhT, bsum,
            W1T, b1, W2T, b2, xo_out, h_out, c_out):
    si = si_ref[0, 0]
    n = xT_ref.shape[1]
    pos = lax.broadcasted_iota(jnp.int32, (1, n), 1)
    oh = jnp.where(pos == si, 1.0, 0.0)
    m = jnp.concatenate([w_ref[...], oh], axis=0)
    a = lax.dot_general(
        m, xT_ref[...], (((1,), (1,)), ((), ())),
        precision=jax.lax.Precision.HIGHEST,
        preferred_element_type=jnp.float32)
    dotn = lambda a, b: lax.dot_general(
        a, b, (((1,), (0,)), ((), ())),
        preferred_element_type=jnp.float32)
    dott = lambda a, b: lax.dot_general(
        a, b, (((1,), (1,)), ((), ())),
        preferred_element_type=jnp.float32)
    agg = a[0:1, :] + a[1:2, :]
    xsi = a[2:3, :]
    xg = dotn(agg, Wn[...]) + dotn(xsi, Ws[...]) + bg[...]
    xg = jnp.maximum(xg, 0.0)
    gates = dotn(xg, WihT[...]) + dotn(h0[...], WhhT[...]) + bsum[...]
    i = jax.nn.sigmoid(gates[:, 0:64])
    f = jax.nn.sigmoid(gates[:, 64:128])
    g = jnp.tanh(gates[:, 128:192])
    o = jax.nn.sigmoid(gates[:, 192:256])
    c1 = f * c0[...] + i * g
    h1 = o * jnp.tanh(c1)
    xcat = jnp.concatenate([xg, h1], axis=1)
    xo = dott(xcat, W1T[...]) + b1[...]
    xo_out[...] = dott(xo, W2T[...]) + b2[...]
    h_out[...] = h1
    c_out[...] = c1


def kernel(x, edge_index, edge_attr, h, c, state_index,
           W_neigh, W_self, b_gnn, W_ih, W_hh, b_ih, b_hh, W1, b1, W2, b2):
    n, nf = x.shape
    E = edge_index.shape[1]
    H = W_neigh.shape[1]

    si = jnp.asarray(state_index, jnp.int32)
    si_vec = jnp.full((L,), si, jnp.int32)
    attr2 = edge_attr.reshape(E // 128, 128)

    wparts = _sc_call(si_vec, edge_index, attr2, n).reshape(NC, n)

    smem_spec = pl.BlockSpec(memory_space=pltpu.SMEM)
    xo, h1, c1 = pl.pallas_call(
        _tc_all,
        in_specs=[smem_spec] + [pl.BlockSpec()] * 14,
        out_shape=(
            jax.ShapeDtypeStruct((1, 4), jnp.float32),
            jax.ShapeDtypeStruct((1, H), jnp.float32),
            jax.ShapeDtypeStruct((1, H), jnp.float32),
        ),
    )(si.reshape(1, 1), wparts, x.T,
      h.reshape(1, H), c.reshape(1, H),
      W_neigh, W_self, b_gnn.reshape(1, H), W_ih.T, W_hh.T,
      (b_ih + b_hh).reshape(1, 4 * H), W1.T, b1.reshape(1, -1),
      W2.T, b2.reshape(1, -1))

    return (xo, h1.reshape(1, 1, H), c1.reshape(1, 1, H))
